# baseline (speedup 1.0000x reference)
import jax
import jax.numpy as jnp
from jax.experimental import pallas as pl

_N_AGENTS = 100
_N_THREADS = 100
_OBS = 64
_HID = 64
_ROWS = _N_THREADS * _N_AGENTS // 2
_RPT = _N_AGENTS // 2


def _fused_body(x_ref, wemb_ref, bemb_ref, wg0_ref, bg0_ref, wg1_ref,
                bg1_ref, wfc_ref, bfc_ref, out_ref):
    x = x_ref[...]
    h = jnp.dot(x, wemb_ref[...], preferred_element_type=jnp.float32)
    h = h + bemb_ref[...]
    for wg_ref, bg_ref in ((wg0_ref, bg0_ref), (wg1_ref, bg1_ref)):
        g = jnp.dot(h, wg_ref[...], preferred_element_type=jnp.float32)
        s = jnp.sum(g.reshape(_N_THREADS, _RPT, 2 * _HID), axis=1)
        m = (s + jnp.roll(s, _HID, axis=1)) * (1.0 / _N_AGENTS)
        m = jnp.broadcast_to(m[:, None, :], (_N_THREADS, _RPT, 2 * _HID))
        h = jnp.tanh(h + m.reshape(_ROWS, 2 * _HID) + bg_ref[...])
    s = jnp.sum(h.reshape(_N_THREADS, _RPT, 2 * _HID), axis=1)
    v = jnp.dot(s * (1.0 / _N_AGENTS), wfc_ref[...],
                preferred_element_type=jnp.float32)
    out_ref[...] = v + bfc_ref[...]


def _blockdiag2(w):
    z = jnp.zeros_like(w)
    return jnp.concatenate(
        [jnp.concatenate([w, z], axis=1), jnp.concatenate([z, w], axis=1)],
        axis=0)


def kernel(cent_obs, rnn_states, masks, edge_index, W_emb, b_emb, W_g0,
           b_g0, W_g1, b_g1, W_fc, b_fc):
    del masks, edge_index
    x = cent_obs.reshape(_ROWS, 2 * _OBS)
    values = pl.pallas_call(
        _fused_body,
        out_shape=jax.ShapeDtypeStruct((_N_THREADS, 1), jnp.float32),
    )(x,
      _blockdiag2(W_emb), jnp.tile(b_emb, 2).reshape(1, 2 * _HID),
      _blockdiag2(W_g0), jnp.tile(b_g0, 2).reshape(1, 2 * _HID),
      _blockdiag2(W_g1), jnp.tile(b_g1, 2).reshape(1, 2 * _HID),
      jnp.concatenate([W_fc, W_fc], axis=0), b_fc.reshape(1, 1))
    return (values, rnn_states)


# speedup vs baseline: 2.1384x; 2.1384x over previous
import jax
import jax.numpy as jnp
from jax.experimental import pallas as pl
from jax.experimental.pallas import tpu as pltpu

_N_AGENTS = 100
_N_THREADS = 100
_OBS = 64
_HID = 64
_CH = 2 * _HID
_NCHUNK = _N_AGENTS // 2
_INV_N = 1.0 / _N_AGENTS


def _fused_body(x_ref, wemb_ref, bemb_ref, wg0_ref, bg0_ref, wg1_ref,
                bg1_ref, wfc_ref, bfc_ref, out_ref, h_ref):
    f32 = jnp.float32
    z = jnp.zeros((_OBS, _HID), dtype=f32)
    wemb = wemb_ref[...]
    we = jnp.concatenate(
        [jnp.concatenate([wemb, z], axis=1),
         jnp.concatenate([z, wemb], axis=1)], axis=0)
    be = jnp.concatenate([bemb_ref[...], bemb_ref[...]], axis=1)

    s = jnp.zeros((_N_THREADS, _CH), dtype=f32)
    for j in range(_NCHUNK):
        xj = x_ref[:, j * _CH:(j + 1) * _CH]
        hj = jnp.dot(xj, we, preferred_element_type=f32) + be
        h_ref[:, j * _CH:(j + 1) * _CH] = hj
        s = s + hj

    for wg_ref, bg_ref, last in ((wg0_ref, bg0_ref, False),
                                 (wg1_ref, bg1_ref, True)):
        wg = wg_ref[...] * _INV_N
        wg2 = jnp.concatenate([wg, wg], axis=1)
        wg4 = jnp.concatenate([wg2, wg2], axis=0)
        bg = jnp.concatenate([bg_ref[...], bg_ref[...]], axis=1)
        m = jnp.dot(s, wg4, preferred_element_type=f32) + bg
        s = jnp.zeros((_N_THREADS, _CH), dtype=f32)
        for j in range(_NCHUNK):
            hj = jnp.tanh(h_ref[:, j * _CH:(j + 1) * _CH] + m)
            if not last:
                h_ref[:, j * _CH:(j + 1) * _CH] = hj
            s = s + hj

    wfc = wfc_ref[...] * _INV_N
    wfc2 = jnp.concatenate([wfc, wfc], axis=0)
    out_ref[...] = jnp.dot(s, wfc2, preferred_element_type=f32) + bfc_ref[...]


def kernel(cent_obs, rnn_states, masks, edge_index, W_emb, b_emb, W_g0,
           b_g0, W_g1, b_g1, W_fc, b_fc):
    del masks, edge_index
    values = pl.pallas_call(
        _fused_body,
        out_shape=jax.ShapeDtypeStruct((_N_THREADS, 1), jnp.float32),
        scratch_shapes=[pltpu.VMEM((_N_THREADS, _N_AGENTS * _HID),
                                   jnp.float32)],
    )(cent_obs, W_emb, b_emb.reshape(1, _HID), W_g0,
      b_g0.reshape(1, _HID), W_g1, b_g1.reshape(1, _HID), W_fc,
      b_fc.reshape(1, 1))
    return (values, rnn_states)
